# baseline (speedup 1.0000x reference)
import functools

import jax
import jax.numpy as jnp
from jax.experimental import pallas as pl
from jax.experimental.pallas import tpu as pltpu


def _moe_body(x_ref, gw_ref, gb_ref, w_ref, b_ref, out_ref, *, n_experts):
    xb = x_ref[...]
    logits = jnp.dot(xb, gw_ref[...], preferred_element_type=jnp.float32)
    logits = logits + gb_ref[...]
    m = jnp.max(logits, axis=-1, keepdims=True)
    p = jnp.exp(logits - m)
    g = p / jnp.sum(p, axis=-1, keepdims=True)

    acc = jnp.zeros(out_ref.shape, jnp.float32)
    for e in range(n_experts):
        ye = jnp.dot(xb, w_ref[e], preferred_element_type=jnp.float32)
        acc = acc + g[:, e : e + 1] * (ye + b_ref[e][None, :])
    out_ref[...] = acc


def kernel(x, gate_W, gate_b, expert_W, expert_b):
    n_tok, k = x.shape
    n_exp, _, n_out = expert_W.shape

    bm = min(512, n_tok)
    bn = min(256, n_out)
    grid = (n_out // bn, n_tok // bm)

    gb2 = gate_b.reshape(1, n_exp)

    body = functools.partial(_moe_body, n_experts=n_exp)
    return pl.pallas_call(
        body,
        grid=grid,
        in_specs=[
            pl.BlockSpec((bm, k), lambda n, m: (m, 0)),
            pl.BlockSpec((k, n_exp), lambda n, m: (0, 0)),
            pl.BlockSpec((1, n_exp), lambda n, m: (0, 0)),
            pl.BlockSpec((n_exp, k, bn), lambda n, m: (0, 0, n)),
            pl.BlockSpec((n_exp, bn), lambda n, m: (0, n)),
        ],
        out_specs=pl.BlockSpec((bm, bn), lambda n, m: (m, n)),
        out_shape=jax.ShapeDtypeStruct((n_tok, n_out), jnp.float32),
        compiler_params=pltpu.CompilerParams(
            dimension_semantics=("arbitrary", "arbitrary"),
        ),
    )(x, gate_W, gb2, expert_W, expert_b)


# speedup vs baseline: 1.0073x; 1.0073x over previous
import functools

import jax
import jax.numpy as jnp
from jax.experimental import pallas as pl
from jax.experimental.pallas import tpu as pltpu


def _moe_body(x_ref, gw_ref, gb_ref, w_ref, b_ref, out_ref, *, n_experts):
    xb = x_ref[...]
    logits = jnp.dot(xb, gw_ref[...], preferred_element_type=jnp.float32)
    logits = logits + gb_ref[...]
    m = jnp.max(logits, axis=-1, keepdims=True)
    p = jnp.exp(logits - m)
    g = p / jnp.sum(p, axis=-1, keepdims=True)

    acc = jnp.dot(g, b_ref[...], preferred_element_type=jnp.float32)
    for e in range(n_experts):
        ye = jnp.dot(xb, w_ref[e], preferred_element_type=jnp.float32)
        acc = acc + g[:, e : e + 1] * ye
    out_ref[...] = acc


def kernel(x, gate_W, gate_b, expert_W, expert_b):
    n_tok, k = x.shape
    n_exp, _, n_out = expert_W.shape

    bm = min(512, n_tok)
    bn = min(256, n_out)
    grid = (n_out // bn, n_tok // bm)

    gb2 = gate_b.reshape(1, n_exp)

    body = functools.partial(_moe_body, n_experts=n_exp)
    return pl.pallas_call(
        body,
        grid=grid,
        in_specs=[
            pl.BlockSpec((bm, k), lambda n, m: (m, 0)),
            pl.BlockSpec((k, n_exp), lambda n, m: (0, 0)),
            pl.BlockSpec((1, n_exp), lambda n, m: (0, 0)),
            pl.BlockSpec((n_exp, k, bn), lambda n, m: (0, 0, n)),
            pl.BlockSpec((n_exp, bn), lambda n, m: (0, n)),
        ],
        out_specs=pl.BlockSpec((bm, bn), lambda n, m: (m, n)),
        out_shape=jax.ShapeDtypeStruct((n_tok, n_out), jnp.float32),
        compiler_params=pltpu.CompilerParams(
            dimension_semantics=("arbitrary", "arbitrary"),
        ),
    )(x, gate_W, gb2, expert_W, expert_b)


# speedup vs baseline: 1.0745x; 1.0668x over previous
import functools

import jax
import jax.numpy as jnp
from jax.experimental import pallas as pl
from jax.experimental.pallas import tpu as pltpu


def _gate_body(x_ref, gw_ref, gb_ref, g_ref):
    logits = jnp.dot(
        x_ref[...].astype(jnp.bfloat16),
        gw_ref[...].astype(jnp.bfloat16),
        preferred_element_type=jnp.float32,
    )
    logits = logits + gb_ref[...]
    m = jnp.max(logits, axis=-1, keepdims=True)
    p = jnp.exp(logits - m)
    g_ref[...] = p / jnp.sum(p, axis=-1, keepdims=True)


def _moe_body(x_ref, g_ref, w_ref, b_ref, out_ref, *, n_experts):
    xb = x_ref[...].astype(jnp.bfloat16)
    g = g_ref[...]
    acc = jnp.dot(g, b_ref[...], preferred_element_type=jnp.float32)
    for e in range(n_experts):
        ye = jnp.dot(xb, w_ref[e].astype(jnp.bfloat16), preferred_element_type=jnp.float32)
        acc = acc + g[:, e : e + 1] * ye
    out_ref[...] = acc


def kernel(x, gate_W, gate_b, expert_W, expert_b):
    n_tok, k = x.shape
    n_exp, _, n_out = expert_W.shape

    bm = min(512, n_tok)
    bn = min(256, n_out)
    gb2 = gate_b.reshape(1, n_exp)

    g = pl.pallas_call(
        _gate_body,
        grid=(n_tok // bm,),
        in_specs=[
            pl.BlockSpec((bm, k), lambda m: (m, 0)),
            pl.BlockSpec((k, n_exp), lambda m: (0, 0)),
            pl.BlockSpec((1, n_exp), lambda m: (0, 0)),
        ],
        out_specs=pl.BlockSpec((bm, n_exp), lambda m: (m, 0)),
        out_shape=jax.ShapeDtypeStruct((n_tok, n_exp), jnp.float32),
    )(x, gate_W, gb2)

    body = functools.partial(_moe_body, n_experts=n_exp)
    return pl.pallas_call(
        body,
        grid=(n_out // bn, n_tok // bm),
        in_specs=[
            pl.BlockSpec((bm, k), lambda n, m: (m, 0)),
            pl.BlockSpec((bm, n_exp), lambda n, m: (m, 0)),
            pl.BlockSpec((n_exp, k, bn), lambda n, m: (0, 0, n)),
            pl.BlockSpec((n_exp, bn), lambda n, m: (0, n)),
        ],
        out_specs=pl.BlockSpec((bm, bn), lambda n, m: (m, n)),
        out_shape=jax.ShapeDtypeStruct((n_tok, n_out), jnp.float32),
        compiler_params=pltpu.CompilerParams(
            dimension_semantics=("arbitrary", "arbitrary"),
        ),
    )(x, g, expert_W, expert_b)


# speedup vs baseline: 1.1647x; 1.0840x over previous
import functools

import jax
import jax.numpy as jnp
from jax.experimental import pallas as pl
from jax.experimental.pallas import tpu as pltpu


def _gate_body(x_ref, gw_ref, gb_ref, g_ref):
    logits = jnp.dot(
        x_ref[...].astype(jnp.bfloat16),
        gw_ref[...].astype(jnp.bfloat16),
        preferred_element_type=jnp.float32,
    )
    logits = logits + gb_ref[...]
    m = jnp.max(logits, axis=-1, keepdims=True)
    p = jnp.exp(logits - m)
    g_ref[...] = p / jnp.sum(p, axis=-1, keepdims=True)


def _moe_body(x_ref, g_ref, w_ref, b_ref, out_ref, *, n_experts):
    xb = x_ref[...].astype(jnp.bfloat16)
    g = g_ref[...]
    acc = jnp.dot(g, b_ref[...], preferred_element_type=jnp.float32)
    for e in range(n_experts):
        ye = jnp.dot(xb, w_ref[e].astype(jnp.bfloat16), preferred_element_type=jnp.float32)
        acc = acc + g[:, e : e + 1] * ye
    out_ref[...] = acc


def kernel(x, gate_W, gate_b, expert_W, expert_b):
    n_tok, k = x.shape
    n_exp, _, n_out = expert_W.shape

    bm = min(1024, n_tok)
    bn = min(256, n_out)
    gb2 = gate_b.reshape(1, n_exp)

    g = pl.pallas_call(
        _gate_body,
        grid=(n_tok // bm,),
        in_specs=[
            pl.BlockSpec((bm, k), lambda m: (m, 0)),
            pl.BlockSpec((k, n_exp), lambda m: (0, 0)),
            pl.BlockSpec((1, n_exp), lambda m: (0, 0)),
        ],
        out_specs=pl.BlockSpec((bm, n_exp), lambda m: (m, 0)),
        out_shape=jax.ShapeDtypeStruct((n_tok, n_exp), jnp.float32),
    )(x, gate_W, gb2)

    body = functools.partial(_moe_body, n_experts=n_exp)
    return pl.pallas_call(
        body,
        grid=(n_out // bn, n_tok // bm),
        in_specs=[
            pl.BlockSpec((bm, k), lambda n, m: (m, 0)),
            pl.BlockSpec((bm, n_exp), lambda n, m: (m, 0)),
            pl.BlockSpec((n_exp, k, bn), lambda n, m: (0, 0, n)),
            pl.BlockSpec((n_exp, bn), lambda n, m: (0, n)),
        ],
        out_specs=pl.BlockSpec((bm, bn), lambda n, m: (m, n)),
        out_shape=jax.ShapeDtypeStruct((n_tok, n_out), jnp.float32),
        compiler_params=pltpu.CompilerParams(
            dimension_semantics=("arbitrary", "arbitrary"),
        ),
    )(x, g, expert_W, expert_b)
